# Initial kernel scaffold; baseline (speedup 1.0000x reference)
import jax
import jax.numpy as jnp
from jax.experimental import pallas as pl


def kernel(x, ln1_g, ln1_b, noop_W, noop_b, cW1, cb1, cW2, cb2, cW3, cb3, emb, ln2_g, ln2_b, fW1, fb1, fW2, fb2, fW3, fb3):
    raise NotImplementedError("write your pallas kernel here")


# speedup vs baseline: 5.8970x; 5.8970x over previous
import math

import jax
import jax.numpy as jnp
from jax.experimental import pallas as pl

_K = 4
_LOG_F = math.log(128.0)


def _ln_rows(x, g, b, eps=1e-5):
    m = jnp.mean(x, axis=-1, keepdims=True)
    v = jnp.mean((x - m) ** 2, axis=-1, keepdims=True)
    return (x - m) * jax.lax.rsqrt(v + eps) * g + b


def _gelu(x):
    return 0.5 * x * (1.0 + jax.lax.erf(x * (1.0 / math.sqrt(2.0))))


def _coarse_kernel(x_ref, ln1g_ref, ln1b_ref, noopw_ref, noopb_ref,
                   cw1_ref, cb1_ref, cw2_ref, cb2_ref, cw3_ref, cb3_ref,
                   fw1a_ref,
                   noop_ref, t_ref, coarse_ref, idx_ref):
    x = x_ref[...]
    xln = _ln_rows(x, ln1g_ref[...], ln1b_ref[...])
    noop_ref[...] = (jnp.sum(xln * noopw_ref[...], axis=-1, keepdims=True)
                     + noopb_ref[...])
    h = _gelu(jnp.dot(xln, cw1_ref[...], preferred_element_type=jnp.float32)
              + cb1_ref[...])
    h = _gelu(jnp.dot(h, cw2_ref[...], preferred_element_type=jnp.float32)
              + cb2_ref[...])
    coarse = (jnp.dot(h, cw3_ref[...], preferred_element_type=jnp.float32)
              + cb3_ref[...])
    coarse_ref[...] = coarse
    t_ref[...] = jnp.dot(xln, fw1a_ref[...], preferred_element_type=jnp.float32)

    n = coarse.shape[-1]
    iota = jax.lax.broadcasted_iota(jnp.int32, coarse.shape, 1)
    vals = coarse
    for k in range(_K):
        m = jnp.max(vals, axis=-1, keepdims=True)
        idxk = jnp.min(jnp.where(vals == m, iota, n), axis=-1, keepdims=True)
        idx_ref[:, k:k + 1] = idxk
        vals = jnp.where(iota == idxk, jnp.float32(-jnp.inf), vals)


def _fine_kernel(t_ref, idx_ref, emb_ref, ln2g_ref, ln2b_ref,
                 fw1b_ref, fb1_ref, fw2_ref, fb2_ref, fw3_ref, fb3_ref,
                 upd_ref):
    t = t_ref[...]
    bt = t.shape[0]
    ntot = emb_ref.shape[0]
    ftot = fw3_ref.shape[1]
    iota_n = jax.lax.broadcasted_iota(jnp.int32, (bt, ntot), 1)
    for k in range(_K):
        idxk = idx_ref[:, k:k + 1]
        oh = (iota_n == idxk).astype(jnp.float32)
        e = jnp.dot(oh, emb_ref[...], preferred_element_type=jnp.float32)
        e = _ln_rows(e, ln2g_ref[...], ln2b_ref[...])
        h = _gelu(t + jnp.dot(e, fw1b_ref[...],
                              preferred_element_type=jnp.float32)
                  + fb1_ref[...])
        h = _gelu(jnp.dot(h, fw2_ref[...], preferred_element_type=jnp.float32)
                  + fb2_ref[...])
        f = (jnp.dot(h, fw3_ref[...], preferred_element_type=jnp.float32)
             + fb3_ref[...])
        m = jnp.max(f, axis=-1, keepdims=True)
        lse = m + jnp.log(jnp.sum(jnp.exp(f - m), axis=-1, keepdims=True))
        upd_ref[:, k * ftot:(k + 1) * ftot] = f + _LOG_F - lse


def _expand_kernel(noop_ref, coarse_ref, idx_ref, upd_ref, out_ref):
    bt = coarse_ref.shape[0]
    coarse = coarse_ref[...]
    m_iota = jax.lax.broadcasted_iota(jnp.int32, (bt, 2048), 1)
    cw_of_m = (m_iota // 16) % 16
    jj = jax.lax.broadcasted_iota(jnp.int32, (128, 2048), 0)
    mm = jax.lax.broadcasted_iota(jnp.int32, (128, 2048), 1)
    T = (jj == 16 * (mm // 256) + mm % 16).astype(jnp.float32)
    c16 = jax.lax.broadcasted_iota(jnp.int32, (16, 2048), 0)
    m16 = jax.lax.broadcasted_iota(jnp.int32, (16, 2048), 1)
    M16 = (c16 == (m16 // 16) % 16).astype(jnp.float32)

    updbig = []
    chk = []
    cwk = []
    for k in range(_K):
        updk = upd_ref[:, 128 * k:128 * (k + 1)]
        updbig.append(jnp.dot(updk, T, preferred_element_type=jnp.float32))
        idxk = idx_ref[:, k:k + 1]
        chk.append(idxk // 16)
        cwk.append(idxk % 16)

    pieces = [noop_ref[...]]
    for ch in range(16):
        seg = jnp.dot(coarse[:, 16 * ch:16 * (ch + 1)], M16,
                      preferred_element_type=jnp.float32) - _LOG_F
        for k in range(_K):
            mask = (chk[k] == ch) & (cwk[k] == cw_of_m)
            seg = seg + jnp.where(mask, updbig[k], 0.0)
        pieces.append(seg)
    out_ref[...] = jnp.concatenate(pieces, axis=-1)


def _full(w):
    return pl.BlockSpec(w.shape, lambda i: (0,) * w.ndim)


def kernel(x, ln1_g, ln1_b, noop_W, noop_b, cW1, cb1, cW2, cb2, cW3, cb3,
           emb, ln2_g, ln2_b, fW1, fb1, fW2, fb2, fW3, fb3):
    B, C = x.shape
    NTOT = cW3.shape[1]
    FTOT = fW3.shape[1]
    f32 = jnp.float32

    def row(v):
        return v.reshape(1, -1)

    fW1a = fW1[:C]
    fW1b = fW1[C:]

    bt1 = 256
    ins1 = (x, row(ln1_g), row(ln1_b), row(noop_W), row(noop_b),
            cW1, row(cb1), cW2, row(cb2), cW3, row(cb3), fW1a)
    noop, t, coarse, idx = pl.pallas_call(
        _coarse_kernel,
        grid=(B // bt1,),
        in_specs=[pl.BlockSpec((bt1, C), lambda i: (i, 0))]
        + [_full(v) for v in ins1[1:]],
        out_specs=[
            pl.BlockSpec((bt1, 1), lambda i: (i, 0)),
            pl.BlockSpec((bt1, C), lambda i: (i, 0)),
            pl.BlockSpec((bt1, NTOT), lambda i: (i, 0)),
            pl.BlockSpec((bt1, _K), lambda i: (i, 0)),
        ],
        out_shape=[
            jax.ShapeDtypeStruct((B, 1), f32),
            jax.ShapeDtypeStruct((B, C), f32),
            jax.ShapeDtypeStruct((B, NTOT), f32),
            jax.ShapeDtypeStruct((B, _K), jnp.int32),
        ],
    )(*ins1)

    bt2 = 128
    ins2 = (t, idx, emb, row(ln2_g), row(ln2_b),
            fW1b, row(fb1), fW2, row(fb2), fW3, row(fb3))
    upd = pl.pallas_call(
        _fine_kernel,
        grid=(B // bt2,),
        in_specs=[
            pl.BlockSpec((bt2, C), lambda i: (i, 0)),
            pl.BlockSpec((bt2, _K), lambda i: (i, 0)),
        ] + [_full(v) for v in ins2[2:]],
        out_specs=pl.BlockSpec((bt2, _K * FTOT), lambda i: (i, 0)),
        out_shape=jax.ShapeDtypeStruct((B, _K * FTOT), f32),
    )(*ins2)

    bt3 = 32
    out = pl.pallas_call(
        _expand_kernel,
        grid=(B // bt3,),
        in_specs=[
            pl.BlockSpec((bt3, 1), lambda i: (i, 0)),
            pl.BlockSpec((bt3, NTOT), lambda i: (i, 0)),
            pl.BlockSpec((bt3, _K), lambda i: (i, 0)),
            pl.BlockSpec((bt3, _K * FTOT), lambda i: (i, 0)),
        ],
        out_specs=pl.BlockSpec((bt3, 1 + NTOT * FTOT), lambda i: (i, 0)),
        out_shape=jax.ShapeDtypeStruct((B, 1 + NTOT * FTOT), f32),
    )(noop, coarse, idx, upd)
    return out


# speedup vs baseline: 6.4541x; 1.0945x over previous
import math

import jax
import jax.numpy as jnp
from jax.experimental import pallas as pl

_K = 4
_LOG_F = math.log(128.0)


def _ln_rows(x, g, b, eps=1e-5):
    m = jnp.mean(x, axis=-1, keepdims=True)
    v = jnp.mean((x - m) ** 2, axis=-1, keepdims=True)
    return (x - m) * jax.lax.rsqrt(v + eps) * g + b


def _gelu(x):
    return 0.5 * x * (1.0 + jax.lax.erf(x * (1.0 / math.sqrt(2.0))))


def _coarse_kernel(x_ref, ln1g_ref, ln1b_ref, noopw_ref, noopb_ref,
                   cw1_ref, cb1_ref, cw2_ref, cb2_ref, cw3_ref, cb3_ref,
                   fw1a_ref,
                   noop_ref, t_ref, coarse_ref, idx_ref):
    x = x_ref[...]
    xln = _ln_rows(x, ln1g_ref[...], ln1b_ref[...])
    noop_ref[...] = (jnp.sum(xln * noopw_ref[...], axis=-1, keepdims=True)
                     + noopb_ref[...])
    h = _gelu(jnp.dot(xln, cw1_ref[...], preferred_element_type=jnp.float32)
              + cb1_ref[...])
    h = _gelu(jnp.dot(h, cw2_ref[...], preferred_element_type=jnp.float32)
              + cb2_ref[...])
    coarse = (jnp.dot(h, cw3_ref[...], preferred_element_type=jnp.float32)
              + cb3_ref[...])
    coarse_ref[...] = coarse
    t_ref[...] = jnp.dot(xln.astype(jnp.bfloat16), fw1a_ref[...],
                         preferred_element_type=jnp.float32)

    n = coarse.shape[-1]
    iota = jax.lax.broadcasted_iota(jnp.int32, coarse.shape, 1)
    vals = coarse
    for k in range(_K):
        m = jnp.max(vals, axis=-1, keepdims=True)
        idxk = jnp.min(jnp.where(vals == m, iota, n), axis=-1, keepdims=True)
        idx_ref[:, k:k + 1] = idxk
        vals = jnp.where(iota == idxk, jnp.float32(-jnp.inf), vals)


def _fine_kernel(t_ref, idx_ref, emb_ref, ln2g_ref, ln2b_ref,
                 fw1b_ref, fb1_ref, fw2_ref, fb2_ref, fw3_ref, fb3_ref,
                 upd_ref):
    t = t_ref[...]
    bt = t.shape[0]
    ntot = emb_ref.shape[0]
    ftot = fw3_ref.shape[1]
    iota_n = jax.lax.broadcasted_iota(jnp.int32, (bt, ntot), 1)
    for k in range(_K):
        idxk = idx_ref[:, k:k + 1]
        oh = (iota_n == idxk).astype(jnp.bfloat16)
        e = jnp.dot(oh, emb_ref[...], preferred_element_type=jnp.float32)
        e = _ln_rows(e, ln2g_ref[...], ln2b_ref[...])
        h = _gelu(t + jnp.dot(e.astype(jnp.bfloat16), fw1b_ref[...],
                              preferred_element_type=jnp.float32)
                  + fb1_ref[...])
        h = _gelu(jnp.dot(h.astype(jnp.bfloat16), fw2_ref[...],
                          preferred_element_type=jnp.float32)
                  + fb2_ref[...])
        f = (jnp.dot(h.astype(jnp.bfloat16), fw3_ref[...],
                     preferred_element_type=jnp.float32)
             + fb3_ref[...])
        m = jnp.max(f, axis=-1, keepdims=True)
        lse = m + jnp.log(jnp.sum(jnp.exp(f - m), axis=-1, keepdims=True))
        upd_ref[:, k * ftot:(k + 1) * ftot] = f + _LOG_F - lse


def _expand_kernel(noop_ref, coarse_ref, idx_ref, upd_ref, out_ref):
    bt = coarse_ref.shape[0]
    coarse = coarse_ref[...]
    jj = jax.lax.broadcasted_iota(jnp.int32, (128, 2048), 0)
    mm = jax.lax.broadcasted_iota(jnp.int32, (128, 2048), 1)
    T = (jj == 16 * (mm // 256) + mm % 16).astype(jnp.float32)
    c16 = jax.lax.broadcasted_iota(jnp.int32, (16, 2048), 0)
    m16 = jax.lax.broadcasted_iota(jnp.int32, (16, 2048), 1)
    M16 = (c16 == (m16 // 16) % 16).astype(jnp.float32)

    M16h = (c16 == (m16 // 16) % 16).astype(jnp.bfloat16)

    updbig = []
    for k in range(_K):
        updk = upd_ref[:, 128 * k:128 * (k + 1)]
        updbig.append(jnp.dot(updk, T, preferred_element_type=jnp.float32))
    iota_n = jax.lax.broadcasted_iota(jnp.int32, (bt, 256), 1)
    sstack = jnp.concatenate(
        [(iota_n == idx_ref[:, k:k + 1]).astype(jnp.bfloat16)
         for k in range(_K)], axis=0)

    pieces = [noop_ref[...]]
    for ch in range(16):
        seg = jnp.dot(coarse[:, 16 * ch:16 * (ch + 1)], M16,
                      preferred_element_type=jnp.float32) - _LOG_F
        sexp = jnp.dot(sstack[:, 16 * ch:16 * (ch + 1)], M16h,
                       preferred_element_type=jnp.float32)
        for k in range(_K):
            seg = seg + sexp[k * bt:(k + 1) * bt] * updbig[k]
        pieces.append(seg)
    out_ref[...] = jnp.concatenate(pieces, axis=-1)


def _full(w):
    return pl.BlockSpec(w.shape, lambda i: (0,) * w.ndim)


def kernel(x, ln1_g, ln1_b, noop_W, noop_b, cW1, cb1, cW2, cb2, cW3, cb3,
           emb, ln2_g, ln2_b, fW1, fb1, fW2, fb2, fW3, fb3):
    B, C = x.shape
    NTOT = cW3.shape[1]
    FTOT = fW3.shape[1]
    f32 = jnp.float32

    def row(v):
        return v.reshape(1, -1)

    bf16 = jnp.bfloat16
    fW1a = fW1[:C].astype(bf16)
    fW1b = fW1[C:].astype(bf16)
    fW2h = fW2.astype(bf16)
    fW3h = fW3.astype(bf16)
    embh = emb.astype(bf16)

    bt1 = 256
    ins1 = (x, row(ln1_g), row(ln1_b), row(noop_W), row(noop_b),
            cW1, row(cb1), cW2, row(cb2), cW3, row(cb3), fW1a)
    noop, t, coarse, idx = pl.pallas_call(
        _coarse_kernel,
        grid=(B // bt1,),
        in_specs=[pl.BlockSpec((bt1, C), lambda i: (i, 0))]
        + [_full(v) for v in ins1[1:]],
        out_specs=[
            pl.BlockSpec((bt1, 1), lambda i: (i, 0)),
            pl.BlockSpec((bt1, C), lambda i: (i, 0)),
            pl.BlockSpec((bt1, NTOT), lambda i: (i, 0)),
            pl.BlockSpec((bt1, _K), lambda i: (i, 0)),
        ],
        out_shape=[
            jax.ShapeDtypeStruct((B, 1), f32),
            jax.ShapeDtypeStruct((B, C), f32),
            jax.ShapeDtypeStruct((B, NTOT), f32),
            jax.ShapeDtypeStruct((B, _K), jnp.int32),
        ],
    )(*ins1)

    bt2 = 256
    ins2 = (t, idx, embh, row(ln2_g), row(ln2_b),
            fW1b, row(fb1), fW2h, row(fb2), fW3h, row(fb3))
    upd = pl.pallas_call(
        _fine_kernel,
        grid=(B // bt2,),
        in_specs=[
            pl.BlockSpec((bt2, C), lambda i: (i, 0)),
            pl.BlockSpec((bt2, _K), lambda i: (i, 0)),
        ] + [_full(v) for v in ins2[2:]],
        out_specs=pl.BlockSpec((bt2, _K * FTOT), lambda i: (i, 0)),
        out_shape=jax.ShapeDtypeStruct((B, _K * FTOT), f32),
    )(*ins2)

    bt3 = 64
    out = pl.pallas_call(
        _expand_kernel,
        grid=(B // bt3,),
        in_specs=[
            pl.BlockSpec((bt3, 1), lambda i: (i, 0)),
            pl.BlockSpec((bt3, NTOT), lambda i: (i, 0)),
            pl.BlockSpec((bt3, _K), lambda i: (i, 0)),
            pl.BlockSpec((bt3, _K * FTOT), lambda i: (i, 0)),
        ],
        out_specs=pl.BlockSpec((bt3, 1 + NTOT * FTOT), lambda i: (i, 0)),
        out_shape=jax.ShapeDtypeStruct((B, 1 + NTOT * FTOT), f32),
    )(noop, coarse, idx, upd)
    return out
